# baseline (speedup 1.0000x reference)
import functools

import jax
import jax.numpy as jnp
from jax import lax
from jax.experimental import pallas as pl
from jax.experimental.pallas import tpu as pltpu
from jax.experimental.pallas import tpu_sc as plsc

B = 16
S = 16
V = 100000
P = 200
P_PAD = 208
VH = V // 2
NEG = -1e30

@functools.cache
def _sc_build_mask_call():
    mesh = plsc.VectorSubcoreMesh(core_axis_name="c", subcore_axis_name="s")
    return pl.kernel(
        _sc_build_mask,
        mesh=mesh,
        out_type=jax.ShapeDtypeStruct((B * V,), jnp.float32),
        scratch_types=[
            pltpu.VMEM((VH,), jnp.float32),
            pltpu.VMEM((P_PAD,), jnp.int32),
        ],
        compiler_params=pltpu.CompilerParams(needs_layout_passes=False),
    )


def _sc_build_mask(seq_hbm, mask_hbm, buf, seq_row):
    cid = lax.axis_index("c")
    sid = lax.axis_index("s")
    wid = sid * 2 + cid
    b = wid // 2
    half = wid % 2
    base_v = half * VH

    def _zero(i, carry):
        buf[pl.ds(i * 16, 16)] = jnp.zeros((16,), jnp.float32)
        return carry

    lax.fori_loop(0, VH // 16, _zero, 0, unroll=8)

    pltpu.sync_copy(seq_hbm.at[pl.ds(b * P_PAD, P_PAD)], seq_row)

    ones = jnp.ones((16,), jnp.float32)

    def _scatter(k, carry):
        ids = seq_row[pl.ds(k * 16, 16)]
        valid = (ids != 0) & (ids >= base_v) & (ids < base_v + VH)
        local = jnp.where(valid, ids - base_v, 0)
        plsc.store_scatter(buf, [local], ones, mask=valid)
        return carry

    lax.fori_loop(0, P_PAD // 16, _scatter, 0, unroll=True)

    pltpu.sync_copy(buf, mask_hbm.at[pl.ds(wid * VH, VH)])


VC = 25088
NV = 4


def _tc_body(logits_ref, mask_ref, out_ref, m_ref, d_ref, n_ref):
    iv = pl.program_id(1)

    @pl.when(iv == 0)
    def _init():
        m_ref[...] = jnp.full((S, 1), NEG, jnp.float32)
        d_ref[...] = jnp.zeros((S, 1), jnp.float32)
        n_ref[...] = jnp.zeros((S, 1), jnp.float32)

    def _step(x, mk):
        cm = jnp.max(x, axis=1, keepdims=True)
        m_prev = m_ref[...]
        m_new = jnp.maximum(m_prev, cm)
        scale = jnp.exp(m_prev - m_new)
        e = jnp.exp(x - m_new)
        rhs = jnp.concatenate([mk, jnp.ones((VC, 1), jnp.float32)], axis=1)
        sums = jax.lax.dot_general(
            e, rhs, (((1,), (0,)), ((), ())),
            preferred_element_type=jnp.float32)
        d_new = d_ref[...] * scale + sums[:, 1:2]
        n_new = n_ref[...] * scale + sums[:, 0:1]
        m_ref[...] = m_new
        d_ref[...] = d_new
        n_ref[...] = n_new
        return n_new, d_new

    @pl.when(iv < NV - 1)
    def _fast():
        _step(logits_ref[0], mask_ref[0])

    @pl.when(iv == NV - 1)
    def _edge():
        cols = lax.broadcasted_iota(jnp.int32, (1, VC), 1) + iv * VC
        x = jnp.where(cols < V, logits_ref[0], NEG)
        rows = lax.broadcasted_iota(jnp.int32, (VC, 1), 0) + iv * VC
        mk = jnp.where(rows < V, mask_ref[0], 0.0)
        n_new, d_new = _step(x, mk)
        out_ref[0] = n_new / d_new


_tc_call = pl.pallas_call(
    _tc_body,
    grid=(B, NV),
    in_specs=[
        pl.BlockSpec((1, S, VC), lambda b, iv: (b, 0, iv)),
        pl.BlockSpec((1, VC, 1), lambda b, iv: (b, iv, 0)),
    ],
    out_specs=pl.BlockSpec((1, S, 1), lambda b, iv: (b, 0, 0)),
    out_shape=jax.ShapeDtypeStruct((B, S, 1), jnp.float32),
    scratch_shapes=[
        pltpu.VMEM((S, 1), jnp.float32),
        pltpu.VMEM((S, 1), jnp.float32),
        pltpu.VMEM((S, 1), jnp.float32),
    ],
)


def kernel(logits, penalty_sequence):
    seq = penalty_sequence.astype(jnp.int32)
    seq_p = jnp.pad(seq, ((0, 0), (0, P_PAD - P)))
    mask = _sc_build_mask_call()(seq_p.reshape(-1))
    mask3 = mask.reshape(B, V, 1)
    out = _tc_call(logits, mask3)
    return out.reshape(B, S)


# speedup vs baseline: 12.7091x; 12.7091x over previous
import functools

import jax
import jax.numpy as jnp
from jax import lax
from jax.experimental import pallas as pl
from jax.experimental.pallas import tpu as pltpu
from jax.experimental.pallas import tpu_sc as plsc

B = 16
S = 16
V = 100000
P = 200
P_PAD = 208
VH = V // 2
NEG = -1e30

@functools.cache
def _sc_build_mask_call():
    mesh = plsc.VectorSubcoreMesh(core_axis_name="c", subcore_axis_name="s")
    return pl.kernel(
        _sc_build_mask,
        mesh=mesh,
        out_type=jax.ShapeDtypeStruct((B * V,), jnp.float32),
        scratch_types=[
            pltpu.VMEM((VH,), jnp.float32),
            pltpu.VMEM((P_PAD,), jnp.int32),
        ],
        compiler_params=pltpu.CompilerParams(needs_layout_passes=False),
    )


def _sc_build_mask(seq_hbm, mask_hbm, buf, seq_row):
    cid = lax.axis_index("c")
    sid = lax.axis_index("s")
    wid = sid * 2 + cid
    b = wid // 2
    half = wid % 2
    base_v = half * VH

    def _zero(i, carry):
        buf[pl.ds(i * 16, 16)] = jnp.zeros((16,), jnp.float32)
        return carry

    lax.fori_loop(0, VH // 16, _zero, 0, unroll=8)

    pltpu.sync_copy(seq_hbm.at[pl.ds(b * P_PAD, P_PAD)], seq_row)

    ones = jnp.ones((16,), jnp.float32)

    def _scatter(k, carry):
        ids = seq_row[pl.ds(k * 16, 16)]
        valid = (ids != 0) & (ids >= base_v) & (ids < base_v + VH)
        local = jnp.where(valid, ids - base_v, 0)
        plsc.store_scatter(buf, [local], ones, mask=valid)
        return carry

    lax.fori_loop(0, P_PAD // 16, _scatter, 0, unroll=True)

    pltpu.sync_copy(buf, mask_hbm.at[pl.ds(wid * VH, VH)])


VC = 100352


def _tc_body(logits_ref, mask_ref, out_ref):
    cols = lax.broadcasted_iota(jnp.int32, (1, VC), 1)
    valid = cols < V
    x = jnp.where(valid, logits_ref[0], NEG)
    m = jnp.max(x, axis=1, keepdims=True)
    e = jnp.exp(x - m)
    mk = jnp.where(valid, mask_ref[0], 0.0)
    d = jnp.sum(e, axis=1, keepdims=True)
    n = jnp.sum(e * mk, axis=1, keepdims=True)
    out_ref[0] = n / d


_tc_call = pl.pallas_call(
    _tc_body,
    grid=(B,),
    in_specs=[
        pl.BlockSpec((1, S, VC), lambda b: (b, 0, 0)),
        pl.BlockSpec((1, 1, VC), lambda b: (b, 0, 0)),
    ],
    out_specs=pl.BlockSpec((1, S, 1), lambda b: (b, 0, 0)),
    out_shape=jax.ShapeDtypeStruct((B, S, 1), jnp.float32),
)


def kernel(logits, penalty_sequence):
    seq = penalty_sequence.astype(jnp.int32)
    seq_p = jnp.pad(seq, ((0, 0), (0, P_PAD - P)))
    mask = _sc_build_mask_call()(seq_p.reshape(-1))
    mask3 = mask.reshape(B, 1, V)
    out = _tc_call(logits, mask3)
    return out.reshape(B, S)
